# Initial kernel scaffold; baseline (speedup 1.0000x reference)
import jax
import jax.numpy as jnp
from jax.experimental import pallas as pl


def kernel(x, edge_index, edge_attr, W_self, W_neigh, bias):
    raise NotImplementedError("write your pallas kernel here")


# speedup vs baseline: 6.1869x; 6.1869x over previous
import functools

import jax
import jax.numpy as jnp
from jax import lax
from jax.experimental import pallas as pl
from jax.experimental.pallas import tpu as pltpu
from jax.experimental.pallas import tpu_sc as plsc

N_NODES = 10000
N_EDGES = 320000
DIM = 128

N_PAD = 10240
ROWS_PER_TILE = 640
NUM_CORES = 2
NUM_SUBCORES = 16
NUM_WORKERS = NUM_CORES * NUM_SUBCORES
EDGES_PER_WORKER = N_EDGES // NUM_WORKERS
CHUNK = 200
ROUNDS = EDGES_PER_WORKER // CHUNK


def _sc_scatter(dst_i32, edge_attr, zrow, zdeg, ones):
    mesh = plsc.VectorSubcoreMesh(core_axis_name="c", subcore_axis_name="s")

    @functools.partial(
        pl.kernel,
        mesh=mesh,
        out_type=[
            jax.ShapeDtypeStruct((NUM_CORES, N_PAD, DIM), jnp.float32),
            jax.ShapeDtypeStruct((NUM_CORES, N_PAD), jnp.float32),
        ],
        scratch_types=[
            pltpu.VMEM((CHUNK,), jnp.int32),
            pltpu.VMEM((CHUNK, DIM), jnp.float32),
            pltpu.VMEM((CHUNK,), jnp.float32),
            pltpu.VMEM((ROWS_PER_TILE,), jnp.float32),
            pltpu.VMEM_SHARED((N_PAD, DIM), jnp.float32),
            pltpu.VMEM_SHARED((N_PAD,), jnp.float32),
        ],
    )
    def scatter_kernel(dst_hbm, attr_hbm, zrow_hbm, zdeg_hbm, ones_hbm,
                       agg_out, deg_out, idx_v, rows_v, ones_v, degc_v,
                       agg_s, deg_s):
        c = lax.axis_index("c")
        s = lax.axis_index("s")
        wid = c * NUM_SUBCORES + s

        pieces = [(0, CHUNK), (CHUNK, CHUNK), (2 * CHUNK, CHUNK),
                  (3 * CHUNK, ROWS_PER_TILE - 3 * CHUNK)]

        row0 = pl.multiple_of(s * ROWS_PER_TILE, 8)
        pltpu.sync_copy(zrow_hbm, rows_v)
        pltpu.sync_copy(zdeg_hbm, degc_v)
        for off, sz in pieces:
            r = pl.multiple_of(row0 + off, 8)
            pltpu.sync_copy(rows_v.at[pl.ds(0, sz)], agg_s.at[pl.ds(r, sz)])
        pltpu.sync_copy(degc_v, deg_s.at[pl.ds(row0, ROWS_PER_TILE)])
        pltpu.sync_copy(ones_hbm, ones_v)
        plsc.subcore_barrier()

        base = wid * EDGES_PER_WORKER

        def body(j, carry):
            e0 = pl.multiple_of(base + j * CHUNK, 8)
            pltpu.sync_copy(dst_hbm.at[pl.ds(e0, CHUNK)], idx_v)
            pltpu.sync_copy(attr_hbm.at[pl.ds(e0, CHUNK)], rows_v)
            pltpu.sync_copy(rows_v, agg_s.at[idx_v], add=True)
            pltpu.sync_copy(ones_v, deg_s.at[idx_v], add=True)
            return carry

        lax.fori_loop(0, ROUNDS, body, 0)
        plsc.subcore_barrier()

        for off, sz in pieces:
            r = pl.multiple_of(row0 + off, 8)
            pltpu.sync_copy(agg_s.at[pl.ds(r, sz)], rows_v.at[pl.ds(0, sz)])
            pltpu.sync_copy(rows_v.at[pl.ds(0, sz)], agg_out.at[c, pl.ds(r, sz)])
        pltpu.sync_copy(deg_s.at[pl.ds(row0, ROWS_PER_TILE)], degc_v)
        pltpu.sync_copy(degc_v, deg_out.at[c, pl.ds(row0, ROWS_PER_TILE)])

    return scatter_kernel(dst_i32, edge_attr, zrow, zdeg, ones)


def _tc_combine_body(x_ref, agg_ref, deg_ref, ws_ref, wn_ref, b_ref, o_ref):
    x = x_ref[...]
    agg = agg_ref[0] + agg_ref[1]
    deg = jnp.maximum(deg_ref[0] + deg_ref[1], 1.0)
    aggm = agg / deg
    out_self = lax.dot_general(x, ws_ref[...], (((1,), (1,)), ((), ())),
                               preferred_element_type=jnp.float32)
    out_neigh = lax.dot_general(aggm, wn_ref[...], (((1,), (1,)), ((), ())),
                                preferred_element_type=jnp.float32)
    o_ref[...] = out_self + out_neigh + b_ref[...]


def _tc_combine(x, agg, deg, W_self, W_neigh, bias):
    blk = 2000
    grid = (N_NODES // blk,)
    return pl.pallas_call(
        _tc_combine_body,
        grid=grid,
        in_specs=[
            pl.BlockSpec((blk, DIM), lambda i: (i, 0)),
            pl.BlockSpec((NUM_CORES, blk, DIM), lambda i: (0, i, 0)),
            pl.BlockSpec((NUM_CORES, blk, 1), lambda i: (0, i, 0)),
            pl.BlockSpec((DIM, DIM), lambda i: (0, 0)),
            pl.BlockSpec((DIM, DIM), lambda i: (0, 0)),
            pl.BlockSpec((DIM,), lambda i: (0,)),
        ],
        out_specs=pl.BlockSpec((blk, DIM), lambda i: (i, 0)),
        out_shape=jax.ShapeDtypeStruct((N_NODES, DIM), jnp.float32),
    )(x, agg, deg, W_self, W_neigh, bias)


def kernel(x, edge_index, edge_attr, W_self, W_neigh, bias):
    dst = edge_index[1].astype(jnp.int32)
    zrow = jnp.zeros((CHUNK, DIM), jnp.float32)
    zdeg = jnp.zeros((ROWS_PER_TILE,), jnp.float32)
    ones = jnp.ones((CHUNK,), jnp.float32)
    agg, deg = _sc_scatter(dst, edge_attr, zrow, zdeg, ones)
    deg = deg.reshape(NUM_CORES, N_PAD, 1)
    return _tc_combine(x, agg, deg, W_self, W_neigh, bias)
